# baseline (speedup 1.0000x reference)
import functools

import jax
import jax.numpy as jnp
from jax import lax
from jax.experimental import pallas as pl
from jax.experimental.pallas import tpu as pltpu
from jax.experimental.pallas import tpu_sc as plsc

_V = 10000
_D = 10000
_DM = 9984
_TW = 128
_TR = _D - _DM
_HA = 4992
_HB = _D - _HA
_B = 2048
_NC = 2
_NS = 16
_NW = _NC * _NS
_BPW = _B // _NW
_K = 8
_NCHUNK = _BPW // _K


@functools.partial(
    pl.kernel,
    out_type=jax.ShapeDtypeStruct((_B, _D), jnp.float32),
    mesh=plsc.VectorSubcoreMesh(core_axis_name="c", subcore_axis_name="s"),
    scratch_types=[
        pltpu.VMEM((_BPW,), jnp.int32),
        pltpu.VMEM((_K, _HA), jnp.float32),
        pltpu.VMEM((_K, _HB), jnp.float32),
        pltpu.VMEM((_K, _TW), jnp.float32),
        pltpu.SemaphoreType.DMA,
        pltpu.SemaphoreType.DMA,
        pltpu.SemaphoreType.DMA,
        pltpu.SemaphoreType.DMA,
        pltpu.SemaphoreType.DMA,
    ],
)
def _gather_rows(sims_hbm, tail_hbm, idx_hbm, out_hbm, idx_v, buf_a, buf_b,
                 tail_v, gsem_a, gsem_b, osem_a, osem_b, tsem):
    wid = lax.axis_index("s") * _NC + lax.axis_index("c")
    base = wid * _BPW
    pltpu.sync_copy(idx_hbm.at[pl.ds(base, _BPW)], idx_v)

    def idx_c(c):
        return idx_v.at[pl.ds(c * _K, _K)]

    def start_a(c):
        return pltpu.async_copy(
            sims_hbm.at[idx_c(c), pl.ds(0, _HA)], buf_a, gsem_a
        )

    def start_b(c):
        return pltpu.async_copy(
            sims_hbm.at[idx_c(c), pl.ds(_HA, _DM - _HA)],
            buf_b.at[:, pl.ds(0, _DM - _HA)], gsem_b,
        )

    def start_t(c):
        return pltpu.async_copy(tail_hbm.at[idx_c(c)], tail_v, tsem)

    ga, gb, th = start_a(0), start_b(0), start_t(0)
    for c in range(_NCHUNK):
        rows = pl.ds(base + c * _K, _K)
        ga.wait()
        oa = pltpu.async_copy(buf_a, out_hbm.at[rows, pl.ds(0, _HA)], osem_a)
        gb.wait()
        th.wait()
        for r in range(_K):
            buf_b[r, pl.ds(_HB - _TR, _TR)] = tail_v[r, pl.ds(_TW - _TR, _TR)]
        ob = pltpu.async_copy(
            buf_b, out_hbm.at[rows, pl.ds(_HA, _HB)], osem_b
        )
        oa.wait()
        if c + 1 < _NCHUNK:
            ga = start_a(c + 1)
        ob.wait()
        if c + 1 < _NCHUNK:
            gb = start_b(c + 1)
            th = start_t(c + 1)


def kernel(x, sims):
    idx = x.reshape(-1).astype(jnp.int32)
    tail = lax.slice(sims, (0, _D - _TW), (_V, _D))
    out = _gather_rows(sims, tail, idx)
    return out.reshape(x.shape[0], x.shape[1], _V)
